# baseline (speedup 1.0000x reference)
import functools

import jax
import jax.numpy as jnp
from jax.experimental import pallas as pl

B, S, H, D = 4, 8192, 64, 6
N = B * S

_MARKET_VEC = (0.5, -1.0, 0.8, 0.6, 1.5, 0.4)
_RISK_VEC = (0.5, -0.8, 0.6, 1.0, 1.2, 0.5)


def _fused_kernel(x_ref, wext_ref, mvrisk_ref, noise_ref,
                  preds_ref, assign_ref, probs_ref):
    x = x_ref[...]
    wext = wext_ref[...]
    stats = jnp.dot(x, wext, preferred_element_type=jnp.float32)

    logits = stats[:, 0:D] + mvrisk_ref[...]
    m = stats[:, D:D + 1]
    m4 = stats[:, D + 1:D + 2]
    m6 = stats[:, D + 2:D + 3]
    m8 = stats[:, D + 3:D + 4]
    m610 = stats[:, D + 4:D + 5]

    sumsq = jnp.sum(x * x, axis=-1, keepdims=True)
    var = (sumsq * (1.0 / H) - m * m) * (H / (H - 1.0))
    std = jnp.sqrt(jnp.maximum(var, 0.0))

    lmax = jnp.max(logits, axis=-1, keepdims=True)
    z = jnp.exp((logits - lmax) * (1.0 / 0.6))
    probs_ref[...] = z / jnp.sum(z, axis=-1, keepdims=True)

    iota = jax.lax.broadcasted_iota(jnp.int32, logits.shape, 1)
    idx = jnp.min(jnp.where(logits == lmax, iota, D), axis=-1, keepdims=True)
    assign_ref[...] = idx

    sig = jax.nn.sigmoid(m)
    d0 = jnp.tanh(m4) * (1.0 + std)
    d1 = sig * 0.3 - 0.15
    d2 = m6 * 0.8 + jnp.sin(m610 * 3.14159) * 0.4
    d3 = jnp.tanh(m8) * 0.9 + noise_ref[...]
    d4 = jnp.maximum(m, 0.0) ** 1.2 + std * 2.5 - 0.5
    d5 = sig * 0.4 + jnp.tanh(std) * 0.2

    preds = jnp.where(idx == 0, d0, 0.0)
    preds = jnp.where(idx == 1, d1, preds)
    preds = jnp.where(idx == 2, d2, preds)
    preds = jnp.where(idx == 3, d3, preds)
    preds = jnp.where(idx == 4, d4, preds)
    preds = jnp.where(idx == 5, d5, preds)
    preds_ref[...] = preds


@functools.partial(jax.jit, static_argnames=("interpret",))
def kernel(sequence_embeddings, market_volatility, risk_factors, W, b,
           interpret=False):
    x = sequence_embeddings.reshape(N, H)

    eye = jnp.zeros((H, 5), dtype=jnp.float32)
    col = jnp.arange(H)[:, None]
    sel = jnp.concatenate([
        (col < H).astype(jnp.float32) / H,
        (col < 4).astype(jnp.float32) / 4.0,
        (col < 6).astype(jnp.float32) / 6.0,
        (col < 8).astype(jnp.float32) / 8.0,
        ((col >= 6) & (col < 10)).astype(jnp.float32) / 4.0,
    ], axis=1)
    del eye
    wext = jnp.concatenate(
        [W.T, sel, jnp.zeros((H, 16 - D - 5), jnp.float32)], axis=1)

    market_vec = jnp.array(_MARKET_VEC, dtype=jnp.float32)
    risk_vec = jnp.array(_RISK_VEC, dtype=jnp.float32)
    mvrisk = (b[None, :]
              + market_volatility.reshape(N, 1) * (market_vec * 0.3)[None, :]
              + risk_factors.reshape(N, D) * (risk_vec * 0.3)[None, :])

    noise = jax.random.normal(jax.random.key(42), (N, 1), jnp.float32) * 0.05

    TB = 4096
    grid = (N // TB,)
    preds, assign, probs = pl.pallas_call(
        _fused_kernel,
        grid=grid,
        in_specs=[
            pl.BlockSpec((TB, H), lambda i: (i, 0)),
            pl.BlockSpec((H, 16), lambda i: (0, 0)),
            pl.BlockSpec((TB, D), lambda i: (i, 0)),
            pl.BlockSpec((TB, 1), lambda i: (i, 0)),
        ],
        out_specs=[
            pl.BlockSpec((TB, 1), lambda i: (i, 0)),
            pl.BlockSpec((TB, 1), lambda i: (i, 0)),
            pl.BlockSpec((TB, D), lambda i: (i, 0)),
        ],
        out_shape=[
            jax.ShapeDtypeStruct((N, 1), jnp.float32),
            jax.ShapeDtypeStruct((N, 1), jnp.int32),
            jax.ShapeDtypeStruct((N, D), jnp.float32),
        ],
        interpret=interpret,
    )(x, wext, mvrisk, noise)

    return (preds.reshape(B, S, 1),
            assign.reshape(B, S),
            probs.reshape(B, S, D))


# speedup vs baseline: 4.8357x; 4.8357x over previous
import functools

import jax
import jax.numpy as jnp
from jax.experimental import pallas as pl

B, S, H, D = 4, 8192, 64, 6
N = B * S

_MARKET_VEC = (0.5, -1.0, 0.8, 0.6, 1.5, 0.4)
_RISK_VEC = (0.5, -0.8, 0.6, 1.0, 1.2, 0.5)


def _fused_kernel(x_ref, wext_ref, b_ref, mv_ref, risk_ref, noise_ref,
                  preds_ref, assign_ref, probs_ref):
    x = x_ref[...]
    xt = x.T
    wext = wext_ref[...]
    stats = jnp.dot(wext, xt, preferred_element_type=jnp.float32)

    def _dvec(vals):
        di = jax.lax.broadcasted_iota(jnp.int32, (D, 1), 0)
        out = jnp.full((D, 1), vals[0] * 0.3, jnp.float32)
        for k in range(1, D):
            out = jnp.where(di == k, vals[k] * 0.3, out)
        return out

    market_vec = _dvec(_MARKET_VEC)
    risk_vec = _dvec(_RISK_VEC)
    logits = (stats[0:D, :] + b_ref[...]
              + mv_ref[...] * market_vec
              + risk_ref[...].T * risk_vec)
    m = stats[D:D + 1, :]
    m4 = stats[D + 1:D + 2, :]
    m6 = stats[D + 2:D + 3, :]
    m8 = stats[D + 3:D + 4, :]
    m610 = stats[D + 4:D + 5, :]

    sumsq = jnp.sum(xt * xt, axis=0, keepdims=True)
    var = (sumsq * (1.0 / H) - m * m) * (H / (H - 1.0))
    std = jnp.sqrt(jnp.maximum(var, 0.0))

    lmax = jnp.max(logits, axis=0, keepdims=True)
    z = jnp.exp((logits - lmax) * (1.0 / 0.6))
    probs_ref[...] = z / jnp.sum(z, axis=0, keepdims=True)

    iota = jax.lax.broadcasted_iota(jnp.int32, logits.shape, 0)
    idx = jnp.min(jnp.where(logits == lmax, iota, D), axis=0, keepdims=True)
    assign_ref[...] = idx

    t3 = jnp.tanh(jnp.concatenate([m4, m8, std], axis=0))
    sig = jax.nn.sigmoid(m)
    d0 = t3[0:1, :] * (1.0 + std)
    d1 = sig * 0.3 - 0.15
    d2 = m6 * 0.8 + jnp.sin(m610 * 3.14159) * 0.4
    d3 = t3[1:2, :] * 0.9 + noise_ref[...]
    d4 = jnp.maximum(m, 0.0) ** 1.2 + std * 2.5 - 0.5
    d5 = sig * 0.4 + t3[2:3, :] * 0.2

    preds = jnp.where(idx == 0, d0, 0.0)
    preds = jnp.where(idx == 1, d1, preds)
    preds = jnp.where(idx == 2, d2, preds)
    preds = jnp.where(idx == 3, d3, preds)
    preds = jnp.where(idx == 4, d4, preds)
    preds_ref[...] = jnp.where(idx == 5, d5, preds)


@functools.partial(jax.jit, static_argnames=("interpret",))
def kernel(sequence_embeddings, market_volatility, risk_factors, W, b,
           interpret=False):
    x = sequence_embeddings.reshape(N, H)

    col = jnp.arange(H)[None, :]
    sel = jnp.concatenate([
        (col < H).astype(jnp.float32) / H,
        (col < 4).astype(jnp.float32) / 4.0,
        (col < 6).astype(jnp.float32) / 6.0,
        (col < 8).astype(jnp.float32) / 8.0,
        ((col >= 6) & (col < 10)).astype(jnp.float32) / 4.0,
        jnp.zeros((1, H), jnp.float32),
    ], axis=0)
    wext = jnp.concatenate([W, sel], axis=0)

    noise = jax.random.normal(jax.random.key(42), (1, N), jnp.float32) * 0.05

    TB = 4096
    grid = (N // TB,)
    preds, assign, probs = pl.pallas_call(
        _fused_kernel,
        grid=grid,
        in_specs=[
            pl.BlockSpec((TB, H), lambda i: (i, 0)),
            pl.BlockSpec((12, H), lambda i: (0, 0)),
            pl.BlockSpec((D, 1), lambda i: (0, 0)),
            pl.BlockSpec((1, TB), lambda i: (0, i)),
            pl.BlockSpec((TB, D), lambda i: (i, 0)),
            pl.BlockSpec((1, TB), lambda i: (0, i)),
        ],
        out_specs=[
            pl.BlockSpec((1, TB), lambda i: (0, i)),
            pl.BlockSpec((1, TB), lambda i: (0, i)),
            pl.BlockSpec((D, TB), lambda i: (0, i)),
        ],
        out_shape=[
            jax.ShapeDtypeStruct((1, N), jnp.float32),
            jax.ShapeDtypeStruct((1, N), jnp.int32),
            jax.ShapeDtypeStruct((D, N), jnp.float32),
        ],
        interpret=interpret,
    )(x, wext, b.reshape(D, 1), market_volatility.reshape(1, N),
      risk_factors.reshape(N, D), noise)

    return (preds.reshape(B, S, 1),
            assign.reshape(B, S),
            probs.T.reshape(B, S, D))
